# baseline (speedup 1.0000x reference)
import functools

import jax
import jax.numpy as jnp
from jax import lax
from jax.experimental import pallas as pl
from jax.experimental.pallas import tpu as pltpu
from jax.experimental.pallas import tpu_sc as plsc

VOCAB = 1000000
MAXLEN = 200
EMBED = 64
BATCH = 4096

NUM_CORES = 2
NUM_SUBCORES = 16
LANES = 16
NUM_WORKERS = NUM_CORES * NUM_SUBCORES

ROWS = BATCH * MAXLEN
ROWS_PER_W = ROWS // NUM_WORKERS
SEQS_PER_W = BATCH // NUM_WORKERS
CHUNK_SEQS = 2
CHUNK_ROWS = CHUNK_SEQS * MAXLEN


def _sc_embed(x_flat, token_table, pos_table):
    mesh = plsc.VectorSubcoreMesh(core_axis_name="c", subcore_axis_name="s")

    @functools.partial(
        pl.kernel,
        out_type=jax.ShapeDtypeStruct((ROWS, EMBED), jnp.float32),
        mesh=mesh,
        compiler_params=pltpu.CompilerParams(use_tc_tiling_on_sc=False),
        scratch_types=[
            pltpu.VMEM((CHUNK_ROWS,), jnp.int32),
            pltpu.VMEM((CHUNK_ROWS, EMBED), jnp.float32),
            pltpu.VMEM((MAXLEN, EMBED), jnp.float32),
            pltpu.SemaphoreType.DMA,
        ],
    )
    def k(x_hbm, tok_hbm, pos_hbm, out_hbm, idx_v, rows_v, pos_v, sem):
        wid = lax.axis_index("s") * NUM_CORES + lax.axis_index("c")
        base = wid * ROWS_PER_W
        pltpu.sync_copy(pos_hbm, pos_v)

        @pl.loop(0, ROWS_PER_W, step=CHUNK_ROWS)
        def _(off):
            pltpu.sync_copy(x_hbm.at[pl.ds(base + off, CHUNK_ROWS)], idx_v)
            pltpu.async_copy(tok_hbm.at[idx_v], rows_v, sem).wait()

            @pl.loop(0, CHUNK_SEQS)
            def _(s):
                @pl.loop(0, MAXLEN)
                def _(r):
                    for c in range(EMBED // LANES):
                        plsc.addupdate(
                            rows_v.at[s * MAXLEN + r, pl.ds(c * LANES, LANES)],
                            pos_v[r, pl.ds(c * LANES, LANES)],
                        )

            pltpu.sync_copy(rows_v, out_hbm.at[pl.ds(base + off, CHUNK_ROWS)])

    return k(x_flat, token_table, pos_table)


def kernel(x, token_table, pos_table):
    x_flat = x.reshape(-1).astype(jnp.int32)
    out = _sc_embed(x_flat, token_table, pos_table)
    return out.reshape(BATCH, MAXLEN, EMBED)
